# Initial kernel scaffold; baseline (speedup 1.0000x reference)
import jax
import jax.numpy as jnp
from jax.experimental import pallas as pl


def kernel(x_req, x_code, x_reason, ei_0, ei_1, ei_2, ei_3, eli, conv0_kqvw_req, conv0_kqvb_req, conv0_outw_req, conv0_outb_req, conv0_skip_req, conv0_kqvw_code, conv0_kqvb_code, conv0_outw_code, conv0_outb_code, conv0_skip_code, conv0_kqvw_reason, conv0_kqvb_reason, conv0_outw_reason, conv0_outb_reason, conv0_skip_reason, conv0_arel_0, conv0_mrel_0, conv0_prel_0, conv0_arel_1, conv0_mrel_1, conv0_prel_1, conv0_arel_2, conv0_mrel_2, conv0_prel_2, conv0_arel_3, conv0_mrel_3, conv0_prel_3, bns0_gamma, bns0_beta, conv1_kqvw_req, conv1_kqvb_req, conv1_outw_req, conv1_outb_req, conv1_skip_req, conv1_kqvw_code, conv1_kqvb_code, conv1_outw_code, conv1_outb_code, conv1_skip_code, conv1_kqvw_reason, conv1_kqvb_reason, conv1_outw_reason, conv1_outb_reason, conv1_skip_reason, conv1_arel_0, conv1_mrel_0, conv1_prel_0, conv1_arel_1, conv1_mrel_1, conv1_prel_1, conv1_arel_2, conv1_mrel_2, conv1_prel_2, conv1_arel_3, conv1_mrel_3, conv1_prel_3, bns1_gamma, bns1_beta, bn2_gamma, bn2_beta, fc1_w, fc1_b, fc2_w, fc2_b, fc3_w, fc3_b, fc4_w, fc4_b, fc5_w, fc5_b, bn3_gamma, bn3_beta, bn4_gamma, bn4_beta, bn5_gamma, bn5_beta, bn6_gamma, bn6_beta):
    raise NotImplementedError("write your pallas kernel here")


# speedup vs baseline: 3.2035x; 3.2035x over previous
import jax
import jax.numpy as jnp
from jax.experimental import pallas as pl
from jax.experimental.pallas import tpu as pltpu

_EPS = 1e-5
_H = 8
_DH = 8
_HID = 64
_PROJW = 320

_ETS = ((0, 1), (1, 0), (0, 2), (2, 0))


def _gelu(x):
    return 0.5 * x * (1.0 + jax.lax.erf(x * (2.0 ** -0.5)))


def _norm_act(y, gamma, beta):
    mu = jnp.mean(y, axis=0, keepdims=True)
    d = y - mu
    inv = jax.lax.rsqrt(jnp.mean(d * d, axis=0, keepdims=True) + _EPS)
    return jnp.maximum(d * inv * gamma + beta, 0.0)


def _mm_bias_kernel(x_ref, w_ref, b_ref, o_ref):
    o_ref[0] = (
        jnp.dot(x_ref[0], w_ref[0], preferred_element_type=jnp.float32)
        + b_ref[0]
    )


def _gelu_out_bn_kernel(agg_ref, w_ref, b_ref, g_ref, be_ref, o_ref):
    a = _gelu(agg_ref[0])
    y = jnp.dot(a, w_ref[0], preferred_element_type=jnp.float32) + b_ref[0]
    o_ref[0] = _norm_act(y, g_ref[...], be_ref[...])


def _gelu_out_skip_bn_kernel(agg_ref, xs_ref, w_ref, b_ref, gate_ref,
                             g_ref, be_ref, o_ref):
    a = _gelu(agg_ref[0])
    y = jnp.dot(a, w_ref[0], preferred_element_type=jnp.float32) + b_ref[0]
    gate = gate_ref[0, 0, 0]
    y = gate * y + (1.0 - gate) * xs_ref[0]
    o_ref[0] = _norm_act(y, g_ref[...], be_ref[...])


def _head_kernel(h_ref, pool_ref, g2_ref, b2_ref,
                 w1_ref, c1_ref, g3_ref, b3_ref,
                 w2_ref, c2_ref, g4_ref, b4_ref,
                 w3_ref, c3_ref, g5_ref, b5_ref,
                 w4_ref, c4_ref, g6_ref, b6_ref,
                 w5_ref, c5_ref, o_ref):
    x = jnp.dot(h_ref[0], pool_ref[...], preferred_element_type=jnp.float32)
    x = _norm_act(x, g2_ref[...], b2_ref[...])
    for w, c, g, b in ((w1_ref, c1_ref, g3_ref, b3_ref),
                       (w2_ref, c2_ref, g4_ref, b4_ref),
                       (w3_ref, c3_ref, g5_ref, b5_ref),
                       (w4_ref, c4_ref, g6_ref, b6_ref)):
        x = _norm_act(
            jnp.dot(x, w[...], preferred_element_type=jnp.float32) + c[...],
            g[...], b[...])
    o_ref[0] = (
        jnp.dot(x, w5_ref[...], preferred_element_type=jnp.float32) + c5_ref[...]
    )


def _stacked_matmul(xs, w, b, tile=512):
    t, n, cin = xs.shape
    cout = w.shape[2]
    return pl.pallas_call(
        _mm_bias_kernel,
        out_shape=jax.ShapeDtypeStruct((t, n, cout), jnp.float32),
        grid=(t, n // tile),
        in_specs=[
            pl.BlockSpec((1, tile, cin), lambda i, j: (i, j, 0)),
            pl.BlockSpec((1, cin, cout), lambda i, j: (i, 0, 0)),
            pl.BlockSpec((1, 1, cout), lambda i, j: (i, 0, 0)),
        ],
        out_specs=pl.BlockSpec((1, tile, cout), lambda i, j: (i, j, 0)),
        compiler_params=pltpu.CompilerParams(
            dimension_semantics=("parallel", "parallel")),
    )(xs, w, b)


def _out_proj(agg, w, b, gamma, beta, x_skip=None, gates=None):
    t, n, c = agg.shape
    full = lambda i: (i, 0, 0)
    shared = lambda i: (0, 0)
    if x_skip is None:
        body, extra, especs = _gelu_out_bn_kernel, (), ()
    else:
        body = _gelu_out_skip_bn_kernel
        extra = (x_skip, gates.reshape(t, 1, 1))
        especs = (pl.BlockSpec((1, n, c), full), pl.BlockSpec((1, 1, 1), full))
    return pl.pallas_call(
        body,
        out_shape=jax.ShapeDtypeStruct((t, n, c), jnp.float32),
        grid=(t,),
        in_specs=[pl.BlockSpec((1, n, c), full)] + list(especs[:1]) + [
            pl.BlockSpec((1, c, c), full),
            pl.BlockSpec((1, 1, c), full),
        ] + list(especs[1:]) + [
            pl.BlockSpec((1, c), shared),
            pl.BlockSpec((1, c), shared),
        ],
        out_specs=pl.BlockSpec((1, n, c), full),
        compiler_params=pltpu.CompilerParams(
            dimension_semantics=("parallel",)),
    )(agg, *extra[:1], w, b.reshape(t, 1, c), *extra[1:],
      gamma.reshape(1, c), beta.reshape(1, c))


def _mlp_head(h, head_params):
    t, n, c = h.shape
    pool = jnp.repeat(jnp.eye(c // 2, dtype=jnp.float32), 2, axis=0) * 0.5
    row = lambda v: v.reshape(1, -1)
    args = [pool] + [row(a) if a.ndim == 1 else a for a in head_params]
    specs = [pl.BlockSpec((1, n, c), lambda i: (i, 0, 0))]
    specs += [pl.BlockSpec(a.shape, lambda i: (0, 0)) for a in args]
    return pl.pallas_call(
        _head_kernel,
        out_shape=jax.ShapeDtypeStruct((t, n, 1), jnp.float32),
        grid=(t,),
        in_specs=specs,
        out_specs=pl.BlockSpec((1, n, 1), lambda i: (i, 0, 0)),
        compiler_params=pltpu.CompilerParams(
            dimension_semantics=("parallel",)),
    )(h, *args)


def _bdiag(m):
    return jax.scipy.linalg.block_diag(*[m[i] for i in range(m.shape[0])])


def _fold(kqvw, kqvb, rels):
    wk, wq, wv = kqvw[:, :_HID], kqvw[:, _HID:2 * _HID], kqvw[:, 2 * _HID:]
    bk, bq, bv = kqvb[:_HID], kqvb[_HID:2 * _HID], kqvb[2 * _HID:]
    wcols, bcols = [wq], [bq]
    for a_rel, m_rel in rels:
        a, m = _bdiag(a_rel), _bdiag(m_rel)
        wcols += [wk @ a, wv @ m]
        bcols += [bk @ a, bv @ m]
    w = jnp.concatenate(wcols, axis=1)
    bvec = jnp.concatenate(bcols, axis=0)
    pad = _PROJW - w.shape[1]
    if pad:
        w = jnp.pad(w, ((0, 0), (0, pad)))
        bvec = jnp.pad(bvec, (0, pad))
    return w, bvec


def _layer_weights(kqvws, kqvbs, arels, mrels):
    ws, bs = [], []
    for ti in range(3):
        rels = [(arels[e], mrels[e]) for e in range(4) if _ETS[e][0] == ti]
        w, b = _fold(kqvws[ti], kqvbs[ti], rels)
        ws.append(w)
        bs.append(b)
    return jnp.stack(ws), jnp.stack(bs)[:, None, :]


def _attention(proj, eis, prels):
    n = proj.shape[1]
    scale = float(_DH) ** -0.5
    col = {}
    for ti in range(3):
        c = _HID
        for e in range(4):
            if _ETS[e][0] == ti:
                col[e] = c
                c += 2 * _HID
    aggs = []
    for dst_t in range(3):
        alphas, msgs, dsts = [], [], []
        for e in range(4):
            src_t, d_t = _ETS[e]
            if d_t != dst_t:
                continue
            src, dst = eis[e][0], eis[e][1]
            c = col[e]
            kv = proj[src_t, :, c:c + 2 * _HID]
            k = kv[:, :_HID].reshape(n, _H, _DH)
            v = kv[:, _HID:].reshape(n, _H, _DH)
            q = proj[dst_t, :, :_HID].reshape(n, _H, _DH)
            alpha = (q[dst] * k[src]).sum(-1) * prels[e] * scale
            alphas.append(alpha)
            msgs.append(v[src])
            dsts.append(dst)
        alpha = jnp.concatenate(alphas, axis=0)
        msg = jnp.concatenate(msgs, axis=0)
        dst = jnp.concatenate(dsts, axis=0)
        ex = jnp.exp(alpha - jnp.max(alpha, axis=0))
        payload = jnp.concatenate(
            [(msg * ex[:, :, None]).reshape(-1, _HID), ex], axis=1)
        seg = jax.ops.segment_sum(payload, dst, num_segments=n)
        agg = seg[:, :_HID].reshape(n, _H, _DH) / (seg[:, _HID:, None] + 1e-16)
        aggs.append(agg.reshape(n, _HID))
    return jnp.stack(aggs)


def kernel(x_req, x_code, x_reason, ei_0, ei_1, ei_2, ei_3, eli,
           conv0_kqvw_req, conv0_kqvb_req, conv0_outw_req, conv0_outb_req, conv0_skip_req,
           conv0_kqvw_code, conv0_kqvb_code, conv0_outw_code, conv0_outb_code, conv0_skip_code,
           conv0_kqvw_reason, conv0_kqvb_reason, conv0_outw_reason, conv0_outb_reason, conv0_skip_reason,
           conv0_arel_0, conv0_mrel_0, conv0_prel_0,
           conv0_arel_1, conv0_mrel_1, conv0_prel_1,
           conv0_arel_2, conv0_mrel_2, conv0_prel_2,
           conv0_arel_3, conv0_mrel_3, conv0_prel_3,
           bns0_gamma, bns0_beta,
           conv1_kqvw_req, conv1_kqvb_req, conv1_outw_req, conv1_outb_req, conv1_skip_req,
           conv1_kqvw_code, conv1_kqvb_code, conv1_outw_code, conv1_outb_code, conv1_skip_code,
           conv1_kqvw_reason, conv1_kqvb_reason, conv1_outw_reason, conv1_outb_reason, conv1_skip_reason,
           conv1_arel_0, conv1_mrel_0, conv1_prel_0,
           conv1_arel_1, conv1_mrel_1, conv1_prel_1,
           conv1_arel_2, conv1_mrel_2, conv1_prel_2,
           conv1_arel_3, conv1_mrel_3, conv1_prel_3,
           bns1_gamma, bns1_beta,
           bn2_gamma, bn2_beta,
           fc1_w, fc1_b, fc2_w, fc2_b, fc3_w, fc3_b, fc4_w, fc4_b, fc5_w, fc5_b,
           bn3_gamma, bn3_beta, bn4_gamma, bn4_beta, bn5_gamma, bn5_beta,
           bn6_gamma, bn6_beta):
    eis = (ei_0, ei_1, ei_2, ei_3)

    xs0 = jnp.stack([x_req, x_code, x_reason])
    w0, b0 = _layer_weights(
        (conv0_kqvw_req, conv0_kqvw_code, conv0_kqvw_reason),
        (conv0_kqvb_req, conv0_kqvb_code, conv0_kqvb_reason),
        (conv0_arel_0, conv0_arel_1, conv0_arel_2, conv0_arel_3),
        (conv0_mrel_0, conv0_mrel_1, conv0_mrel_2, conv0_mrel_3))
    proj0 = _stacked_matmul(xs0, w0, b0)
    agg0 = _attention(proj0, eis,
                      (conv0_prel_0, conv0_prel_1, conv0_prel_2, conv0_prel_3))
    ow0 = jnp.stack([conv0_outw_req, conv0_outw_code, conv0_outw_reason])
    ob0 = jnp.stack([conv0_outb_req, conv0_outb_code, conv0_outb_reason])
    h0 = _out_proj(agg0, ow0, ob0, bns0_gamma, bns0_beta)

    w1, b1 = _layer_weights(
        (conv1_kqvw_req, conv1_kqvw_code, conv1_kqvw_reason),
        (conv1_kqvb_req, conv1_kqvb_code, conv1_kqvb_reason),
        (conv1_arel_0, conv1_arel_1, conv1_arel_2, conv1_arel_3),
        (conv1_mrel_0, conv1_mrel_1, conv1_mrel_2, conv1_mrel_3))
    proj1 = _stacked_matmul(h0, w1, b1)
    agg1 = _attention(proj1, eis,
                      (conv1_prel_0, conv1_prel_1, conv1_prel_2, conv1_prel_3))
    ow1 = jnp.stack([conv1_outw_req, conv1_outw_code, conv1_outw_reason])
    ob1 = jnp.stack([conv1_outb_req, conv1_outb_code, conv1_outb_reason])
    gates = jax.nn.sigmoid(jnp.stack(
        [conv1_skip_req, conv1_skip_code, conv1_skip_reason]))
    h1 = _out_proj(agg1, ow1, ob1, bns1_gamma, bns1_beta,
                   x_skip=h0, gates=gates)

    head_params = (bn2_gamma, bn2_beta,
                   fc1_w, fc1_b, bn3_gamma, bn3_beta,
                   fc2_w, fc2_b, bn4_gamma, bn4_beta,
                   fc3_w, fc3_b, bn5_gamma, bn5_beta,
                   fc4_w, fc4_b, bn6_gamma, bn6_beta,
                   fc5_w, fc5_b)
    out = _mlp_head(h1, head_params)
    a = out[0, :, 0][eli[0]]
    b = out[1, :, 0][eli[1]]
    return jax.nn.sigmoid(a * b)


# speedup vs baseline: 10.9551x; 3.4197x over previous
import jax
import jax.numpy as jnp
from jax.experimental import pallas as pl
from jax.experimental.pallas import tpu as pltpu

_EPS = 1e-5
_H = 8
_DH = 8
_HID = 64
_PROJW = 320

_ETS = ((0, 1), (1, 0), (0, 2), (2, 0))


_ET_ORDER = (1, 3, 0, 2)
_E = 16384
_ETOT = 4 * _E
_TILE_E = 64


def _gelu(x):
    return 0.5 * x * (1.0 + jax.lax.erf(x * (2.0 ** -0.5)))


def _norm_act(y, gamma, beta):
    mu = jnp.mean(y, axis=0, keepdims=True)
    d = y - mu
    inv = jax.lax.rsqrt(jnp.mean(d * d, axis=0, keepdims=True) + _EPS)
    return jnp.maximum(d * inv * gamma + beta, 0.0)


def _mm_bias_kernel(x_ref, w_ref, b_ref, o_ref):
    o_ref[0] = (
        jnp.dot(x_ref[0], w_ref[0], preferred_element_type=jnp.float32)
        + b_ref[0]
    )


def _gelu_out_bn_kernel(agg_ref, w_ref, b_ref, g_ref, be_ref, o_ref):
    a = _gelu(agg_ref[0])
    y = jnp.dot(a, w_ref[0], preferred_element_type=jnp.float32) + b_ref[0]
    o_ref[0] = _norm_act(y, g_ref[...], be_ref[...])


def _gelu_out_skip_bn_kernel(agg_ref, xs_ref, w_ref, b_ref, gate_ref,
                             g_ref, be_ref, o_ref):
    a = _gelu(agg_ref[0])
    y = jnp.dot(a, w_ref[0], preferred_element_type=jnp.float32) + b_ref[0]
    gate = gate_ref[0, 0, 0]
    y = gate * y + (1.0 - gate) * xs_ref[0]
    o_ref[0] = _norm_act(y, g_ref[...], be_ref[...])


def _head_kernel(h_ref, pool_ref, g2_ref, b2_ref,
                 w1_ref, c1_ref, g3_ref, b3_ref,
                 w2_ref, c2_ref, g4_ref, b4_ref,
                 w3_ref, c3_ref, g5_ref, b5_ref,
                 w4_ref, c4_ref, g6_ref, b6_ref,
                 w5_ref, c5_ref, o_ref):
    x = jnp.dot(h_ref[0], pool_ref[...], preferred_element_type=jnp.float32)
    x = _norm_act(x, g2_ref[...], b2_ref[...])
    for w, c, g, b in ((w1_ref, c1_ref, g3_ref, b3_ref),
                       (w2_ref, c2_ref, g4_ref, b4_ref),
                       (w3_ref, c3_ref, g5_ref, b5_ref),
                       (w4_ref, c4_ref, g6_ref, b6_ref)):
        x = _norm_act(
            jnp.dot(x, w[...], preferred_element_type=jnp.float32) + c[...],
            g[...], b[...])
    o_ref[0] = (
        jnp.dot(x, w5_ref[...], preferred_element_type=jnp.float32) + c5_ref[...]
    )


def _edge_gather_kernel(kv_slot_ref, q_slot_ref, q_all_ref, kv_all_ref,
                        pay_ref, q_scr, kv_scr):
    pid = pl.program_id(0)
    half = _ETOT // 2
    r = jax.lax.broadcasted_iota(jnp.int32, (_HID, _H), 0)
    c = jax.lax.broadcasted_iota(jnp.int32, (_HID, _H), 1)
    pool8 = (r // _DH == c).astype(jnp.float32)

    def tile(t, carry):
        base = pid * half + t * _TILE_E
        for u in range(_TILE_E):
            e = base + u
            q_scr[u] = q_all_ref[q_slot_ref[e], 0]
            kv_scr[u] = kv_all_ref[kv_slot_ref[e], 0]
        q = q_scr[...]
        kv = kv_scr[...]
        alpha = jnp.dot(q * kv[:, :_HID], pool8,
                        preferred_element_type=jnp.float32)
        rows = pl.ds(pl.multiple_of(t * _TILE_E, 8), _TILE_E)
        pay_ref[rows, 0:_HID] = kv[:, _HID:]
        pay_ref[rows, _HID:_HID + _H] = alpha
        return carry

    jax.lax.fori_loop(0, half // _TILE_E, tile, 0)


def _edge_gather(q_all, kv_all, kv_slot, q_slot):
    return pl.pallas_call(
        _edge_gather_kernel,
        out_shape=jax.ShapeDtypeStruct((_ETOT, _HID + _H), jnp.float32),
        grid=(2,),
        in_specs=[
            pl.BlockSpec(memory_space=pltpu.MemorySpace.SMEM),
            pl.BlockSpec(memory_space=pltpu.MemorySpace.SMEM),
            pl.BlockSpec((12288, 1, _HID), lambda i: (0, 0, 0)),
            pl.BlockSpec((_ETOT // 4, 1, 2 * _HID), lambda i: (0, 0, 0)),
        ],
        out_specs=pl.BlockSpec((_ETOT // 2, _HID + _H), lambda i: (i, 0)),
        scratch_shapes=[
            pltpu.VMEM((_TILE_E, _HID), jnp.float32),
            pltpu.VMEM((_TILE_E, 2 * _HID), jnp.float32),
        ],
        compiler_params=pltpu.CompilerParams(
            dimension_semantics=("parallel",)),
    )(kv_slot, q_slot, q_all, kv_all)


def _stacked_matmul(xs, w, b, tile=512):
    t, n, cin = xs.shape
    cout = w.shape[2]
    return pl.pallas_call(
        _mm_bias_kernel,
        out_shape=jax.ShapeDtypeStruct((t, n, cout), jnp.float32),
        grid=(t, n // tile),
        in_specs=[
            pl.BlockSpec((1, tile, cin), lambda i, j: (i, j, 0)),
            pl.BlockSpec((1, cin, cout), lambda i, j: (i, 0, 0)),
            pl.BlockSpec((1, 1, cout), lambda i, j: (i, 0, 0)),
        ],
        out_specs=pl.BlockSpec((1, tile, cout), lambda i, j: (i, j, 0)),
        compiler_params=pltpu.CompilerParams(
            dimension_semantics=("parallel", "parallel")),
    )(xs, w, b)


def _out_proj(agg, w, b, gamma, beta, x_skip=None, gates=None):
    t, n, c = agg.shape
    full = lambda i: (i, 0, 0)
    shared = lambda i: (0, 0)
    if x_skip is None:
        body, extra, especs = _gelu_out_bn_kernel, (), ()
    else:
        body = _gelu_out_skip_bn_kernel
        extra = (x_skip, gates.reshape(t, 1, 1))
        especs = (pl.BlockSpec((1, n, c), full), pl.BlockSpec((1, 1, 1), full))
    return pl.pallas_call(
        body,
        out_shape=jax.ShapeDtypeStruct((t, n, c), jnp.float32),
        grid=(t,),
        in_specs=[pl.BlockSpec((1, n, c), full)] + list(especs[:1]) + [
            pl.BlockSpec((1, c, c), full),
            pl.BlockSpec((1, 1, c), full),
        ] + list(especs[1:]) + [
            pl.BlockSpec((1, c), shared),
            pl.BlockSpec((1, c), shared),
        ],
        out_specs=pl.BlockSpec((1, n, c), full),
        compiler_params=pltpu.CompilerParams(
            dimension_semantics=("parallel",)),
    )(agg, *extra[:1], w, b.reshape(t, 1, c), *extra[1:],
      gamma.reshape(1, c), beta.reshape(1, c))


def _mlp_head(h, head_params):
    t, n, c = h.shape
    pool = jnp.repeat(jnp.eye(c // 2, dtype=jnp.float32), 2, axis=0) * 0.5
    row = lambda v: v.reshape(1, -1)
    args = [pool] + [row(a) if a.ndim == 1 else a for a in head_params]
    specs = [pl.BlockSpec((1, n, c), lambda i: (i, 0, 0))]
    specs += [pl.BlockSpec(a.shape, lambda i: (0, 0)) for a in args]
    return pl.pallas_call(
        _head_kernel,
        out_shape=jax.ShapeDtypeStruct((t, n, 1), jnp.float32),
        grid=(t,),
        in_specs=specs,
        out_specs=pl.BlockSpec((1, n, 1), lambda i: (i, 0, 0)),
        compiler_params=pltpu.CompilerParams(
            dimension_semantics=("parallel",)),
    )(h, *args)


def _bdiag(m):
    return jax.scipy.linalg.block_diag(*[m[i] for i in range(m.shape[0])])


def _fold(kqvw, kqvb, rels):
    wk, wq, wv = kqvw[:, :_HID], kqvw[:, _HID:2 * _HID], kqvw[:, 2 * _HID:]
    bk, bq, bv = kqvb[:_HID], kqvb[_HID:2 * _HID], kqvb[2 * _HID:]
    wcols, bcols = [wq], [bq]
    scale = float(_DH) ** -0.5
    for a_rel, m_rel, prel in rels:
        a = _bdiag(a_rel * (prel * scale)[:, None, None])
        m = _bdiag(m_rel)
        wcols += [wk @ a, wv @ m]
        bcols += [bk @ a, bv @ m]
    w = jnp.concatenate(wcols, axis=1)
    bvec = jnp.concatenate(bcols, axis=0)
    pad = _PROJW - w.shape[1]
    if pad:
        w = jnp.pad(w, ((0, 0), (0, pad)))
        bvec = jnp.pad(bvec, (0, pad))
    return w, bvec


def _layer_weights(kqvws, kqvbs, arels, mrels, prels):
    ws, bs = [], []
    for ti in range(3):
        rels = [(arels[e], mrels[e], prels[e])
                for e in range(4) if _ETS[e][0] == ti]
        w, b = _fold(kqvws[ti], kqvbs[ti], rels)
        ws.append(w)
        bs.append(b)
    return jnp.stack(ws), jnp.stack(bs)[:, None, :]


def _kv_cols():
    col = {}
    for ti in range(3):
        c = _HID
        for e in range(4):
            if _ETS[e][0] == ti:
                col[e] = c
                c += 2 * _HID
    return col


def _edge_slots(eis):
    kv_slots, q_slots = [], []
    for p, e in enumerate(_ET_ORDER):
        _, dst_t = _ETS[e]
        kv_slots.append(p * 4096 + eis[e][0])
        q_slots.append(dst_t * 4096 + eis[e][1])
    return jnp.concatenate(kv_slots), jnp.concatenate(q_slots)


def _attention(proj, eis, kv_slot, q_slot):
    n = proj.shape[1]
    col = _kv_cols()
    q_all = proj[:, :, :_HID].reshape(3 * n, 1, _HID)
    kv_all = jnp.concatenate(
        [proj[_ETS[e][0], :, col[e]:col[e] + 2 * _HID] for e in _ET_ORDER],
        axis=0).reshape(4 * n, 1, 2 * _HID)
    pay = _edge_gather(q_all, kv_all, kv_slot, q_slot)
    aggs = []
    bounds = ((0, 2 * _E, (1, 3)), (2 * _E, 3 * _E, (0,)),
              (3 * _E, 4 * _E, (2,)))
    for lo, hi, ets_in in bounds:
        alpha = pay[lo:hi, _HID:]
        msg = pay[lo:hi, :_HID]
        dst = jnp.concatenate([eis[e][1] for e in ets_in], axis=0)
        ex = jnp.exp(alpha - jnp.max(alpha, axis=0))
        payload = jnp.concatenate(
            [(msg.reshape(-1, _H, _DH) * ex[:, :, None]).reshape(-1, _HID),
             ex], axis=1)
        seg = jax.ops.segment_sum(payload, dst, num_segments=n)
        agg = seg[:, :_HID].reshape(n, _H, _DH) / (seg[:, _HID:, None] + 1e-16)
        aggs.append(agg.reshape(n, _HID))
    return jnp.stack(aggs)


def kernel(x_req, x_code, x_reason, ei_0, ei_1, ei_2, ei_3, eli,
           conv0_kqvw_req, conv0_kqvb_req, conv0_outw_req, conv0_outb_req, conv0_skip_req,
           conv0_kqvw_code, conv0_kqvb_code, conv0_outw_code, conv0_outb_code, conv0_skip_code,
           conv0_kqvw_reason, conv0_kqvb_reason, conv0_outw_reason, conv0_outb_reason, conv0_skip_reason,
           conv0_arel_0, conv0_mrel_0, conv0_prel_0,
           conv0_arel_1, conv0_mrel_1, conv0_prel_1,
           conv0_arel_2, conv0_mrel_2, conv0_prel_2,
           conv0_arel_3, conv0_mrel_3, conv0_prel_3,
           bns0_gamma, bns0_beta,
           conv1_kqvw_req, conv1_kqvb_req, conv1_outw_req, conv1_outb_req, conv1_skip_req,
           conv1_kqvw_code, conv1_kqvb_code, conv1_outw_code, conv1_outb_code, conv1_skip_code,
           conv1_kqvw_reason, conv1_kqvb_reason, conv1_outw_reason, conv1_outb_reason, conv1_skip_reason,
           conv1_arel_0, conv1_mrel_0, conv1_prel_0,
           conv1_arel_1, conv1_mrel_1, conv1_prel_1,
           conv1_arel_2, conv1_mrel_2, conv1_prel_2,
           conv1_arel_3, conv1_mrel_3, conv1_prel_3,
           bns1_gamma, bns1_beta,
           bn2_gamma, bn2_beta,
           fc1_w, fc1_b, fc2_w, fc2_b, fc3_w, fc3_b, fc4_w, fc4_b, fc5_w, fc5_b,
           bn3_gamma, bn3_beta, bn4_gamma, bn4_beta, bn5_gamma, bn5_beta,
           bn6_gamma, bn6_beta):
    eis = (ei_0, ei_1, ei_2, ei_3)
    kv_slot, q_slot = _edge_slots(eis)

    xs0 = jnp.stack([x_req, x_code, x_reason])
    w0, b0 = _layer_weights(
        (conv0_kqvw_req, conv0_kqvw_code, conv0_kqvw_reason),
        (conv0_kqvb_req, conv0_kqvb_code, conv0_kqvb_reason),
        (conv0_arel_0, conv0_arel_1, conv0_arel_2, conv0_arel_3),
        (conv0_mrel_0, conv0_mrel_1, conv0_mrel_2, conv0_mrel_3),
        (conv0_prel_0, conv0_prel_1, conv0_prel_2, conv0_prel_3))
    proj0 = _stacked_matmul(xs0, w0, b0)
    agg0 = _attention(proj0, eis, kv_slot, q_slot)
    ow0 = jnp.stack([conv0_outw_req, conv0_outw_code, conv0_outw_reason])
    ob0 = jnp.stack([conv0_outb_req, conv0_outb_code, conv0_outb_reason])
    h0 = _out_proj(agg0, ow0, ob0, bns0_gamma, bns0_beta)

    w1, b1 = _layer_weights(
        (conv1_kqvw_req, conv1_kqvw_code, conv1_kqvw_reason),
        (conv1_kqvb_req, conv1_kqvb_code, conv1_kqvb_reason),
        (conv1_arel_0, conv1_arel_1, conv1_arel_2, conv1_arel_3),
        (conv1_mrel_0, conv1_mrel_1, conv1_mrel_2, conv1_mrel_3),
        (conv1_prel_0, conv1_prel_1, conv1_prel_2, conv1_prel_3))
    proj1 = _stacked_matmul(h0, w1, b1)
    agg1 = _attention(proj1, eis, kv_slot, q_slot)
    ow1 = jnp.stack([conv1_outw_req, conv1_outw_code, conv1_outw_reason])
    ob1 = jnp.stack([conv1_outb_req, conv1_outb_code, conv1_outb_reason])
    gates = jax.nn.sigmoid(jnp.stack(
        [conv1_skip_req, conv1_skip_code, conv1_skip_reason]))
    h1 = _out_proj(agg1, ow1, ob1, bns1_gamma, bns1_beta,
                   x_skip=h0, gates=gates)

    head_params = (bn2_gamma, bn2_beta,
                   fc1_w, fc1_b, bn3_gamma, bn3_beta,
                   fc2_w, fc2_b, bn4_gamma, bn4_beta,
                   fc3_w, fc3_b, bn5_gamma, bn5_beta,
                   fc4_w, fc4_b, bn6_gamma, bn6_beta,
                   fc5_w, fc5_b)
    out = _mlp_head(h1, head_params)
    a = out[0, :, 0][eli[0]]
    b = out[1, :, 0][eli[1]]
    return jax.nn.sigmoid(a * b)
